# baseline (speedup 1.0000x reference)
import functools

import jax
import jax.numpy as jnp
from jax import lax
from jax.experimental import pallas as pl
from jax.experimental.pallas import tpu as pltpu
from jax.experimental.pallas import tpu_sc as plsc

BATCH = 4
SEQ_LEN = 8192
HIDDEN = 768
TOTAL = BATCH * SEQ_LEN
NROWS_TBL = SEQ_LEN + 1
NCORES = 2
NSUB = 16
COLS_PER_CORE = HIDDEN // NCORES
NPASS = 3
COLS_PER_PASS = COLS_PER_CORE // NPASS
ROWS_PER_TILE = TOTAL // NSUB
CHUNK = 128
NBUF = 3
NCHUNKS = ROWS_PER_TILE // CHUNK
LOAD_ROWS = 512
SPM_ROWS = NSUB * LOAD_ROWS + 8


def _make_sc_gather():
    mesh = plsc.VectorSubcoreMesh(core_axis_name="c", subcore_axis_name="s")

    @functools.partial(
        pl.kernel,
        mesh=mesh,
        out_type=jax.ShapeDtypeStruct((TOTAL, HIDDEN), jnp.float32),
        scratch_types=[
            pltpu.VMEM((ROWS_PER_TILE,), jnp.int32),
            pltpu.VMEM((NBUF, CHUNK, COLS_PER_PASS), jnp.float32),
            pltpu.VMEM_SHARED((SPM_ROWS, COLS_PER_PASS), jnp.float32),
            pltpu.SemaphoreType.DMA,
            pltpu.SemaphoreType.DMA,
            pltpu.SemaphoreType.DMA,
        ],
    )
    def sc_gather(table_hbm, tail_hbm, idx_hbm, out_hbm,
                  idx_v, rows_v, spm, gsem, ssem, lsem):
        c = lax.axis_index("c")
        s = lax.axis_index("s")
        rbase = s * ROWS_PER_TILE

        def start_load(p):
            coff = c * COLS_PER_CORE + p * COLS_PER_PASS
            cp = pltpu.async_copy(
                table_hbm.at[pl.ds(s * LOAD_ROWS, LOAD_ROWS),
                             pl.ds(coff, COLS_PER_PASS)],
                spm.at[pl.ds(s * LOAD_ROWS, LOAD_ROWS)],
                lsem,
            )

            @pl.when(s == 0)
            def _():
                pltpu.sync_copy(
                    tail_hbm.at[pl.ds(0, 8), pl.ds(coff, COLS_PER_PASS)],
                    spm.at[pl.ds(NSUB * LOAD_ROWS, 8)],
                )

            return cp

        def start_gather(j, slot):
            return pltpu.async_copy(
                spm.at[idx_v.at[pl.ds(j * CHUNK, CHUNK)]],
                rows_v.at[slot],
                gsem,
            )

        def start_store(j, slot, coff):
            return pltpu.async_copy(
                rows_v.at[slot],
                out_hbm.at[pl.ds(rbase + j * CHUNK, CHUNK),
                           pl.ds(coff, COLS_PER_PASS)],
                ssem,
            )

        load = start_load(0)
        pltpu.sync_copy(idx_hbm.at[pl.ds(rbase, ROWS_PER_TILE)], idx_v)

        LOOKAHEAD = 2
        store_on_slot = [None] * NBUF

        for p in range(NPASS):
            coff = c * COLS_PER_CORE + p * COLS_PER_PASS

            load.wait()
            plsc.subcore_barrier()

            gathers = [None] * NCHUNKS

            def prep(j, p=p):
                slot = (p * NCHUNKS + j) % NBUF
                if store_on_slot[slot] is not None:
                    store_on_slot[slot].wait()
                    store_on_slot[slot] = None
                gathers[j] = start_gather(j, slot)

            for b in range(LOOKAHEAD):
                prep(b)
            for j in range(NCHUNKS):
                slot = (p * NCHUNKS + j) % NBUF
                gathers[j].wait()
                store_on_slot[slot] = start_store(j, slot, coff)
                if j + LOOKAHEAD < NCHUNKS:
                    prep(j + LOOKAHEAD)
            plsc.subcore_barrier()
            if p + 1 < NPASS:
                load = start_load(p + 1)

        for slot in range(NBUF):
            if store_on_slot[slot] is not None:
                store_on_slot[slot].wait()

    return sc_gather


_sc_gather = _make_sc_gather()


@jax.jit
def kernel(x, table):
    tail8 = jnp.broadcast_to(table[NROWS_TBL - 1], (8, HIDDEN))
    out = _sc_gather(table, tail8, x.reshape(TOTAL))
    return out.reshape(BATCH, SEQ_LEN, HIDDEN)
